# baseline (speedup 1.0000x reference)
import functools

import jax
import jax.numpy as jnp
from jax import lax
from jax.experimental import pallas as pl
from jax.experimental.pallas import tpu as pltpu
from jax.experimental.pallas import tpu_sc as plsc

N_TOKENS = 16384
K_WORDS = 8192
EMB = 256
TB = 256
TILE = 128
N_TILES = K_WORDS // TILE
_BOUNDS = (0, 2736, 5472, 8192)
CHUNKS = tuple(
    (_BOUNDS[c], _BOUNDS[c + 1], _BOUNDS[c] // TILE,
     -(-_BOUNDS[c + 1] // TILE))
    for c in range(3))
_SHARED_TILES = tuple(b // TILE for b in _BOUNDS[1:3] if b % TILE != 0)
BIG_I32 = 2 ** 30


def _dist_argmin_kernel(z_ref, znorm_ref, cb_ref, cnorm_ref, idx_ref, loss_ref):
    i = pl.program_id(0)
    zb = z_ref[...].astype(jnp.bfloat16)
    zb_lo, zb_hi = zb[:, :128], zb[:, 128:]
    znorm = znorm_ref[...]

    def tile_d(t):
        cbt = cb_ref[pl.ds(t * TILE, TILE), :]
        mm = (lax.dot_general(zb_lo, cbt[:, :128], (((1,), (1,)), ((), ())),
                              preferred_element_type=jnp.float32)
              + lax.dot_general(zb_hi, cbt[:, 128:], (((1,), (1,)), ((), ())),
                                preferred_element_type=jnp.float32))
        cn = cnorm_ref[t]
        return (znorm + cn) - 2.0 * mm

    gkf = lax.broadcasted_iota(jnp.int32, (TB, TILE), 1).astype(jnp.float32)
    v = jnp.full((TB,), jnp.inf, dtype=jnp.float32)
    bidx = jnp.zeros((TB,), dtype=jnp.int32)
    cache = {}
    for (c0, c1, ta, tz) in CHUNKS:
        m_c = jnp.full((TB,), jnp.inf, dtype=jnp.float32)
        i_c = jnp.zeros((TB,), dtype=jnp.float32)
        for t in range(ta, tz):
            if t in cache:
                dt = cache.pop(t)
            else:
                dt = tile_d(t)
                if t in _SHARED_TILES:
                    cache[t] = dt
            if t * TILE >= c0 and (t + 1) * TILE <= c1:
                dm = dt
            else:
                gk = lax.broadcasted_iota(jnp.int32, (TB, TILE), 1) + t * TILE
                inside = (gk >= c0) & (gk < c1)
                dm = jnp.where(inside, dt, jnp.inf)
            mt = jnp.min(dm, axis=1)
            it = jnp.min(jnp.where(dm == mt[:, None], gkf, jnp.float32(1e9)),
                         axis=1) + jnp.float32(t * TILE)
            take = mt < m_c
            i_c = jnp.where(take, it, i_c)
            m_c = jnp.where(take, mt, m_c)
        i_ci = i_c.astype(jnp.int32)
        take = (m_c < v) | ((m_c == v) & (i_ci < bidx))
        bidx = jnp.where(take, i_ci, bidx)
        v = jnp.where(take, m_c, v)
        v = v.astype(jnp.bfloat16).astype(jnp.float32)

    idx_ref[...] = bidx
    s = jnp.reshape(jnp.sum(v), (1, 1))

    @pl.when(i == 0)
    def _():
        loss_ref[...] = s

    @pl.when(i > 0)
    def _():
        loss_ref[...] = loss_ref[...] + s

    @pl.when(i == (N_TOKENS // TB) - 1)
    def _():
        loss_ref[...] = loss_ref[...] * jnp.float32(1.33 / (N_TOKENS * EMB))


def _dist_argmin(z_flat, znorm, cbb, cnorm3d, interpret=False):
    return pl.pallas_call(
        _dist_argmin_kernel,
        grid=(N_TOKENS // TB,),
        in_specs=[
            pl.BlockSpec((TB, EMB), lambda i: (i, 0)),
            pl.BlockSpec((TB, 1), lambda i: (i, 0)),
            pl.BlockSpec((K_WORDS, EMB), lambda i: (0, 0)),
            pl.BlockSpec((N_TILES, 1, TILE), lambda i: (0, 0, 0)),
        ],
        out_specs=[
            pl.BlockSpec((TB,), lambda i: (i,)),
            pl.BlockSpec((1, 1), lambda i: (0, 0)),
        ],
        out_shape=[
            jax.ShapeDtypeStruct((N_TOKENS,), jnp.int32),
            jax.ShapeDtypeStruct((1, 1), jnp.float32),
        ],
        interpret=interpret,
    )(z_flat, znorm, cbb, cnorm3d)


_GB = 128
_NW = 32


def _sc_gather(codebook, idx):
    mesh = plsc.VectorSubcoreMesh(core_axis_name="c", subcore_axis_name="s")
    b_per_w = N_TOKENS // _NW

    @functools.partial(
        pl.kernel, mesh=mesh,
        out_type=jax.ShapeDtypeStruct((N_TOKENS, EMB), jnp.float32),
        scratch_types=[
            pltpu.VMEM((_GB,), jnp.int32),
            pltpu.VMEM((_GB, EMB), jnp.float32),
            pltpu.SemaphoreType.DMA,
        ],
    )
    def k(table_hbm, idx_hbm, out_hbm, idx_v, rows_v, sem):
        wid = lax.axis_index("s") * 2 + lax.axis_index("c")
        base = wid * b_per_w
        for chn in range(b_per_w // _GB):
            off = base + chn * _GB
            pltpu.sync_copy(idx_hbm.at[pl.ds(off, _GB)], idx_v)
            pltpu.async_copy(table_hbm.at[idx_v], rows_v, sem).wait()
            pltpu.sync_copy(rows_v, out_hbm.at[pl.ds(off, _GB)])

    return k(codebook, idx)


def kernel(z, codebook):
    b, c, h, w = z.shape
    zp = jnp.transpose(z, (0, 2, 3, 1))
    z_flat = zp.reshape(-1, c)
    znorm = jnp.sum(z_flat ** 2, axis=1, keepdims=True)
    cnorm = jnp.sum(codebook ** 2, axis=1)
    idx, loss2d = _dist_argmin(z_flat, znorm, codebook.astype(jnp.bfloat16),
                               cnorm.reshape(N_TILES, 1, TILE))
    zq_flat = _sc_gather(codebook, idx)
    zq = zq_flat.reshape(zp.shape)
    zq_st = zp + lax.stop_gradient(zq - zp)
    out = jnp.transpose(zq_st, (0, 3, 1, 2))
    return out, loss2d[0, 0], idx


# speedup vs baseline: 1.9924x; 1.9924x over previous
import functools

import jax
import jax.numpy as jnp
from jax import lax
from jax.experimental import pallas as pl
from jax.experimental.pallas import tpu as pltpu
from jax.experimental.pallas import tpu_sc as plsc

N_TOKENS = 16384
K_WORDS = 8192
EMB = 256
TB = 256
TILE = 1024
N_TILES = K_WORDS // TILE
_BOUNDS = (0, 2736, 5472, 8192)
CHUNKS = tuple(
    (_BOUNDS[c], _BOUNDS[c + 1], _BOUNDS[c] // TILE,
     -(-_BOUNDS[c + 1] // TILE))
    for c in range(3))
_SHARED_TILES = tuple(b // TILE for b in _BOUNDS[1:3] if b % TILE != 0)
BIG_I32 = 2 ** 30


def _dist_argmin_kernel(z_ref, znorm_ref, cb_ref, cnorm_ref, idx_ref, loss_ref):
    i = pl.program_id(0)
    zb = z_ref[...].astype(jnp.bfloat16)
    zb_lo, zb_hi = zb[:, :128], zb[:, 128:]
    znorm = znorm_ref[...]

    def tile_d(t):
        cbt = cb_ref[pl.ds(t * TILE, TILE), :]
        mm = (lax.dot_general(zb_lo, cbt[:, :128], (((1,), (1,)), ((), ())),
                              preferred_element_type=jnp.float32)
              + lax.dot_general(zb_hi, cbt[:, 128:], (((1,), (1,)), ((), ())),
                                preferred_element_type=jnp.float32))
        cn = cnorm_ref[t]
        return (znorm + cn) - 2.0 * mm

    gkf = lax.broadcasted_iota(jnp.int32, (TB, TILE), 1).astype(jnp.float32)
    v = jnp.full((TB,), jnp.inf, dtype=jnp.float32)
    bidx = jnp.zeros((TB,), dtype=jnp.int32)
    cache = {}
    for (c0, c1, ta, tz) in CHUNKS:
        m_c = jnp.full((TB,), jnp.inf, dtype=jnp.float32)
        i_c = jnp.zeros((TB,), dtype=jnp.float32)
        for t in range(ta, tz):
            if t in cache:
                dt = cache.pop(t)
            else:
                dt = tile_d(t)
                if t in _SHARED_TILES:
                    cache[t] = dt
            if t * TILE >= c0 and (t + 1) * TILE <= c1:
                dm = dt
            else:
                gk = lax.broadcasted_iota(jnp.int32, (TB, TILE), 1) + t * TILE
                inside = (gk >= c0) & (gk < c1)
                dm = jnp.where(inside, dt, jnp.inf)
            mt = jnp.min(dm, axis=1)
            it = jnp.min(jnp.where(dm == mt[:, None], gkf, jnp.float32(1e9)),
                         axis=1) + jnp.float32(t * TILE)
            take = mt < m_c
            i_c = jnp.where(take, it, i_c)
            m_c = jnp.where(take, mt, m_c)
        i_ci = i_c.astype(jnp.int32)
        take = (m_c < v) | ((m_c == v) & (i_ci < bidx))
        bidx = jnp.where(take, i_ci, bidx)
        v = jnp.where(take, m_c, v)
        v = v.astype(jnp.bfloat16).astype(jnp.float32)

    idx_ref[...] = bidx
    s = jnp.reshape(jnp.sum(v), (1, 1))

    @pl.when(i == 0)
    def _():
        loss_ref[...] = s

    @pl.when(i > 0)
    def _():
        loss_ref[...] = loss_ref[...] + s

    @pl.when(i == (N_TOKENS // TB) - 1)
    def _():
        loss_ref[...] = loss_ref[...] * jnp.float32(1.33 / (N_TOKENS * EMB))


def _dist_argmin(z_flat, znorm, cbb, cnorm3d, interpret=False):
    return pl.pallas_call(
        _dist_argmin_kernel,
        grid=(N_TOKENS // TB,),
        in_specs=[
            pl.BlockSpec((TB, EMB), lambda i: (i, 0)),
            pl.BlockSpec((TB, 1), lambda i: (i, 0)),
            pl.BlockSpec((K_WORDS, EMB), lambda i: (0, 0)),
            pl.BlockSpec((N_TILES, 1, TILE), lambda i: (0, 0, 0)),
        ],
        out_specs=[
            pl.BlockSpec((TB,), lambda i: (i,)),
            pl.BlockSpec((1, 1), lambda i: (0, 0)),
        ],
        out_shape=[
            jax.ShapeDtypeStruct((N_TOKENS,), jnp.int32),
            jax.ShapeDtypeStruct((1, 1), jnp.float32),
        ],
        interpret=interpret,
    )(z_flat, znorm, cbb, cnorm3d)


_GB = 128
_NW = 32


def _sc_gather(codebook, idx):
    mesh = plsc.VectorSubcoreMesh(core_axis_name="c", subcore_axis_name="s")
    b_per_w = N_TOKENS // _NW

    @functools.partial(
        pl.kernel, mesh=mesh,
        out_type=jax.ShapeDtypeStruct((N_TOKENS, EMB), jnp.float32),
        scratch_types=[
            pltpu.VMEM((_GB,), jnp.int32),
            pltpu.VMEM((_GB, EMB), jnp.float32),
            pltpu.SemaphoreType.DMA,
        ],
    )
    def k(table_hbm, idx_hbm, out_hbm, idx_v, rows_v, sem):
        wid = lax.axis_index("s") * 2 + lax.axis_index("c")
        base = wid * b_per_w
        for chn in range(b_per_w // _GB):
            off = base + chn * _GB
            pltpu.sync_copy(idx_hbm.at[pl.ds(off, _GB)], idx_v)
            pltpu.async_copy(table_hbm.at[idx_v], rows_v, sem).wait()
            pltpu.sync_copy(rows_v, out_hbm.at[pl.ds(off, _GB)])

    return k(codebook, idx)


def kernel(z, codebook):
    b, c, h, w = z.shape
    zp = jnp.transpose(z, (0, 2, 3, 1))
    z_flat = zp.reshape(-1, c)
    znorm = jnp.sum(z_flat ** 2, axis=1, keepdims=True)
    cnorm = jnp.sum(codebook ** 2, axis=1)
    idx, loss2d = _dist_argmin(z_flat, znorm, codebook.astype(jnp.bfloat16),
                               cnorm.reshape(N_TILES, 1, TILE))
    zq_flat = _sc_gather(codebook, idx)
    zq = zq_flat.reshape(zp.shape)
    zq_st = zp + lax.stop_gradient(zq - zp)
    out = jnp.transpose(zq_st, (0, 3, 1, 2))
    return out, loss2d[0, 0], idx


# speedup vs baseline: 2.1308x; 1.0695x over previous
import functools

import jax
import jax.numpy as jnp
from jax import lax
from jax.experimental import pallas as pl
from jax.experimental.pallas import tpu as pltpu
from jax.experimental.pallas import tpu_sc as plsc

N_TOKENS = 16384
K_WORDS = 8192
EMB = 256
TB = 512
TILE = 512
N_TILES = K_WORDS // TILE
_BOUNDS = (0, 2736, 5472, 8192)
CHUNKS = tuple(
    (_BOUNDS[c], _BOUNDS[c + 1], _BOUNDS[c] // TILE,
     -(-_BOUNDS[c + 1] // TILE))
    for c in range(3))
_SHARED_TILES = tuple(b // TILE for b in _BOUNDS[1:3] if b % TILE != 0)
BIG_I32 = 2 ** 30


def _dist_argmin_kernel(z_ref, znorm_ref, cb_ref, cnorm_ref, idx_ref, loss_ref):
    i = pl.program_id(0)
    zb = z_ref[...].astype(jnp.bfloat16)
    zb_lo, zb_hi = zb[:, :128], zb[:, 128:]
    znorm = znorm_ref[...]

    def tile_d(t):
        cbt = cb_ref[pl.ds(t * TILE, TILE), :]
        mm = (lax.dot_general(zb_lo, cbt[:, :128], (((1,), (1,)), ((), ())),
                              preferred_element_type=jnp.float32)
              + lax.dot_general(zb_hi, cbt[:, 128:], (((1,), (1,)), ((), ())),
                                preferred_element_type=jnp.float32))
        cn = cnorm_ref[t]
        return (znorm + cn) - 2.0 * mm

    gkf = lax.broadcasted_iota(jnp.int32, (TB, TILE), 1).astype(jnp.float32)
    v = jnp.full((TB,), jnp.inf, dtype=jnp.float32)
    bidx = jnp.zeros((TB,), dtype=jnp.int32)
    cache = {}
    for (c0, c1, ta, tz) in CHUNKS:
        m_c = jnp.full((TB,), jnp.inf, dtype=jnp.float32)
        i_c = jnp.zeros((TB,), dtype=jnp.float32)
        for t in range(ta, tz):
            if t in cache:
                dt = cache.pop(t)
            else:
                dt = tile_d(t)
                if t in _SHARED_TILES:
                    cache[t] = dt
            if t * TILE >= c0 and (t + 1) * TILE <= c1:
                dm = dt
            else:
                gk = lax.broadcasted_iota(jnp.int32, (TB, TILE), 1) + t * TILE
                inside = (gk >= c0) & (gk < c1)
                dm = jnp.where(inside, dt, jnp.inf)
            mt = jnp.min(dm, axis=1)
            it = jnp.min(jnp.where(dm == mt[:, None], gkf, jnp.float32(1e9)),
                         axis=1) + jnp.float32(t * TILE)
            take = mt < m_c
            i_c = jnp.where(take, it, i_c)
            m_c = jnp.where(take, mt, m_c)
        i_ci = i_c.astype(jnp.int32)
        take = (m_c < v) | ((m_c == v) & (i_ci < bidx))
        bidx = jnp.where(take, i_ci, bidx)
        v = jnp.where(take, m_c, v)
        v = v.astype(jnp.bfloat16).astype(jnp.float32)

    idx_ref[...] = bidx
    s = jnp.reshape(jnp.sum(v), (1, 1))

    @pl.when(i == 0)
    def _():
        loss_ref[...] = s

    @pl.when(i > 0)
    def _():
        loss_ref[...] = loss_ref[...] + s

    @pl.when(i == (N_TOKENS // TB) - 1)
    def _():
        loss_ref[...] = loss_ref[...] * jnp.float32(1.33 / (N_TOKENS * EMB))


def _dist_argmin(z_flat, znorm, cbb, cnorm3d, interpret=False):
    return pl.pallas_call(
        _dist_argmin_kernel,
        grid=(N_TOKENS // TB,),
        in_specs=[
            pl.BlockSpec((TB, EMB), lambda i: (i, 0)),
            pl.BlockSpec((TB, 1), lambda i: (i, 0)),
            pl.BlockSpec((K_WORDS, EMB), lambda i: (0, 0)),
            pl.BlockSpec((N_TILES, 1, TILE), lambda i: (0, 0, 0)),
        ],
        out_specs=[
            pl.BlockSpec((TB,), lambda i: (i,)),
            pl.BlockSpec((1, 1), lambda i: (0, 0)),
        ],
        out_shape=[
            jax.ShapeDtypeStruct((N_TOKENS,), jnp.int32),
            jax.ShapeDtypeStruct((1, 1), jnp.float32),
        ],
        interpret=interpret,
    )(z_flat, znorm, cbb, cnorm3d)


_GB = 128
_NW = 32


def _sc_gather(codebook, idx):
    mesh = plsc.VectorSubcoreMesh(core_axis_name="c", subcore_axis_name="s")
    b_per_w = N_TOKENS // _NW

    @functools.partial(
        pl.kernel, mesh=mesh,
        out_type=jax.ShapeDtypeStruct((N_TOKENS, EMB), jnp.float32),
        scratch_types=[
            pltpu.VMEM((_GB,), jnp.int32),
            pltpu.VMEM((_GB, EMB), jnp.float32),
            pltpu.SemaphoreType.DMA,
        ],
    )
    def k(table_hbm, idx_hbm, out_hbm, idx_v, rows_v, sem):
        wid = lax.axis_index("s") * 2 + lax.axis_index("c")
        base = wid * b_per_w
        for chn in range(b_per_w // _GB):
            off = base + chn * _GB
            pltpu.sync_copy(idx_hbm.at[pl.ds(off, _GB)], idx_v)
            pltpu.async_copy(table_hbm.at[idx_v], rows_v, sem).wait()
            pltpu.sync_copy(rows_v, out_hbm.at[pl.ds(off, _GB)])

    return k(codebook, idx)


def kernel(z, codebook):
    b, c, h, w = z.shape
    zp = jnp.transpose(z, (0, 2, 3, 1))
    z_flat = zp.reshape(-1, c)
    znorm = jnp.sum(z_flat ** 2, axis=1, keepdims=True)
    cnorm = jnp.sum(codebook ** 2, axis=1)
    idx, loss2d = _dist_argmin(z_flat, znorm, codebook.astype(jnp.bfloat16),
                               cnorm.reshape(N_TILES, 1, TILE))
    zq_flat = _sc_gather(codebook, idx)
    zq = zq_flat.reshape(zp.shape)
    zq_st = zp + lax.stop_gradient(zq - zp)
    out = jnp.transpose(zq_st, (0, 3, 1, 2))
    return out, loss2d[0, 0], idx
